# Initial kernel scaffold; baseline (speedup 1.0000x reference)
import jax
import jax.numpy as jnp
from jax.experimental import pallas as pl


def kernel(x, edge_index, W1, b1, W2, b2, W3, b3):
    raise NotImplementedError("write your pallas kernel here")


# speedup vs baseline: 9.5764x; 9.5764x over previous
import functools

import jax
import jax.numpy as jnp
from jax import lax
from jax.experimental import pallas as pl
from jax.experimental.pallas import tpu as pltpu
from jax.experimental.pallas import tpu_sc as plsc

N = 10000
E = 160000
TILES = 16
NPAD = 10112
RPT = NPAD // TILES

K1 = 80
CH1 = E // TILES // K1
K2 = 40
CH2 = E // 32 // K2

_SELU_ALPHA = 1.6732632423543772
_SELU_SCALE = 1.0507009873554805

_mesh = plsc.VectorSubcoreMesh(
    core_axis_name="c", subcore_axis_name="s", num_cores=2, num_subcores=16
)


@functools.partial(
    pl.kernel,
    out_type=jax.ShapeDtypeStruct((2, NPAD, 128), jnp.float32),
    mesh=_mesh,
    scratch_types=[
        pltpu.VMEM((CH2, K2), jnp.int32),
        pltpu.VMEM((K2, 128), jnp.float32),
        pltpu.VMEM_SHARED((NPAD, 128), jnp.float32),
    ],
)
def _sc_degree(dst_hbm, ones_hbm, zeros_hbm, out_hbm, dstv, onesv, acc):
    c = lax.axis_index("c")
    s = lax.axis_index("s")
    flat = c * TILES + s
    row0 = s * RPT
    pltpu.sync_copy(dst_hbm.at[flat], dstv)
    pltpu.sync_copy(ones_hbm.at[pl.ds(0, K2)], onesv)

    @pl.when(c == 0)
    def _():
        pltpu.sync_copy(ones_hbm.at[pl.ds(row0, RPT)], acc.at[pl.ds(row0, RPT)])

    @pl.when(c == 1)
    def _():
        pltpu.sync_copy(zeros_hbm.at[pl.ds(row0, RPT)], acc.at[pl.ds(row0, RPT)])

    plsc.subcore_barrier()

    def body(j, carry):
        pltpu.sync_copy(onesv, acc.at[dstv.at[j]], add=True)
        return carry

    lax.fori_loop(0, CH2, body, 0)
    plsc.subcore_barrier()
    pltpu.sync_copy(acc.at[pl.ds(row0, RPT)], out_hbm.at[c, pl.ds(row0, RPT)])


@functools.partial(
    pl.kernel,
    out_type=jax.ShapeDtypeStruct((2, NPAD, 128), jnp.float32),
    mesh=_mesh,
    scratch_types=[
        pltpu.VMEM((CH1, K1), jnp.int32),
        pltpu.VMEM((CH1, K1), jnp.int32),
        pltpu.VMEM((K1, 128), jnp.float32),
        pltpu.VMEM_SHARED((NPAD, 128), jnp.float32),
        pltpu.SemaphoreType.DMA,
    ],
)
def _sc_agg1(g_hbm, srcA_hbm, srcB_hbm, dst_hbm, out_hbm, srcv, dstv, buf, acc, sem):
    c = lax.axis_index("c")
    s = lax.axis_index("s")
    row0 = s * RPT

    @pl.when(c == 0)
    def _():
        pltpu.sync_copy(srcA_hbm.at[s], srcv)

    @pl.when(c == 1)
    def _():
        pltpu.sync_copy(srcB_hbm.at[s], srcv)

    pltpu.sync_copy(dst_hbm.at[s], dstv)
    pltpu.sync_copy(g_hbm.at[pl.ds(c * NPAD + row0, RPT)], acc.at[pl.ds(row0, RPT)])
    plsc.subcore_barrier()

    def body(j, carry):
        pltpu.async_copy(g_hbm.at[srcv.at[j]], buf, sem).wait()
        pltpu.sync_copy(buf, acc.at[dstv.at[j]], add=True)
        return carry

    lax.fori_loop(0, CH1, body, 0)
    plsc.subcore_barrier()
    pltpu.sync_copy(acc.at[pl.ds(row0, RPT)], out_hbm.at[c, pl.ds(row0, RPT)])


@functools.partial(
    pl.kernel,
    out_type=jax.ShapeDtypeStruct((2, NPAD, 128), jnp.float32),
    mesh=_mesh,
    scratch_types=[
        pltpu.VMEM((CH2, K2), jnp.int32),
        pltpu.VMEM((CH2, K2), jnp.int32),
        pltpu.VMEM((K2, 128), jnp.float32),
        pltpu.VMEM_SHARED((NPAD, 128), jnp.float32),
        pltpu.SemaphoreType.DMA,
    ],
)
def _sc_agg2(g_hbm, zeros_hbm, src_hbm, dst_hbm, out_hbm, srcv, dstv, buf, acc, sem):
    c = lax.axis_index("c")
    s = lax.axis_index("s")
    flat = c * TILES + s
    row0 = s * RPT
    pltpu.sync_copy(src_hbm.at[flat], srcv)
    pltpu.sync_copy(dst_hbm.at[flat], dstv)

    @pl.when(c == 0)
    def _():
        pltpu.sync_copy(g_hbm.at[pl.ds(row0, RPT)], acc.at[pl.ds(row0, RPT)])

    @pl.when(c == 1)
    def _():
        pltpu.sync_copy(zeros_hbm.at[pl.ds(row0, RPT)], acc.at[pl.ds(row0, RPT)])

    plsc.subcore_barrier()

    def body(j, carry):
        pltpu.async_copy(g_hbm.at[srcv.at[j]], buf, sem).wait()
        pltpu.sync_copy(buf, acc.at[dstv.at[j]], add=True)
        return carry

    lax.fori_loop(0, CH2, body, 0)
    plsc.subcore_barrier()
    pltpu.sync_copy(acc.at[pl.ds(row0, RPT)], out_hbm.at[c, pl.ds(row0, RPT)])


_BR = 632
_GRID = NPAD // _BR


def _tc_dinv_body(d_ref, out_ref):
    d = d_ref[0] + d_ref[1]
    out_ref[...] = jnp.where(d > 0, lax.rsqrt(d), 0.0)


def _tc_dinv(deg2):
    return pl.pallas_call(
        _tc_dinv_body,
        grid=(_GRID,),
        in_specs=[pl.BlockSpec((2, _BR, 128), lambda i: (0, i, 0))],
        out_specs=pl.BlockSpec((_BR, 128), lambda i: (i, 0)),
        out_shape=jax.ShapeDtypeStruct((NPAD, 128), jnp.float32),
    )(deg2)


def _tc_layer1_body(x_ref, w_ref, dinv_ref, out_ref):
    dv = dinv_ref[:, 0:1]
    g = jnp.dot(x_ref[...] * dv, w_ref[...], preferred_element_type=jnp.float32)
    out_ref[0] = g[:, :128]
    out_ref[1] = g[:, 128:]


def _tc_layer1(x, W1, dinv):
    return pl.pallas_call(
        _tc_layer1_body,
        grid=(_GRID,),
        in_specs=[
            pl.BlockSpec((_BR, 256), lambda i: (i, 0)),
            pl.BlockSpec((256, 256), lambda i: (0, 0)),
            pl.BlockSpec((_BR, 128), lambda i: (i, 0)),
        ],
        out_specs=pl.BlockSpec((2, _BR, 128), lambda i: (0, i, 0)),
        out_shape=jax.ShapeDtypeStruct((2, NPAD, 128), jnp.float32),
    )(x, W1, dinv)


def _selu(x):
    return _SELU_SCALE * jnp.where(x > 0, x, _SELU_ALPHA * jnp.exp(x) - _SELU_ALPHA)


def _tc_mid2_body(s_ref, dinv_ref, b_ref, w_ref, out_ref):
    dv = dinv_ref[:, 0:1]
    u0 = dv * _selu(dv * s_ref[0] + b_ref[0:1, :])
    u1 = dv * _selu(dv * s_ref[1] + b_ref[1:2, :])
    g = jnp.dot(u0, w_ref[:128, :], preferred_element_type=jnp.float32)
    g += jnp.dot(u1, w_ref[128:, :], preferred_element_type=jnp.float32)
    out_ref[...] = g


def _tc_mid2(s1, dinv, b2d, W2):
    return pl.pallas_call(
        _tc_mid2_body,
        grid=(_GRID,),
        in_specs=[
            pl.BlockSpec((2, _BR, 128), lambda i: (0, i, 0)),
            pl.BlockSpec((_BR, 128), lambda i: (i, 0)),
            pl.BlockSpec((2, 128), lambda i: (0, 0)),
            pl.BlockSpec((256, 128), lambda i: (0, 0)),
        ],
        out_specs=pl.BlockSpec((_BR, 128), lambda i: (i, 0)),
        out_shape=jax.ShapeDtypeStruct((NPAD, 128), jnp.float32),
    )(s1, dinv, b2d, W2)


def _tc_mid3_body(s_ref, dinv_ref, b_ref, w_ref, out_ref):
    dv = dinv_ref[:, 0:1]
    u = dv * _selu(dv * (s_ref[0] + s_ref[1]) + b_ref[...])
    out_ref[...] = jnp.dot(u, w_ref[...], preferred_element_type=jnp.float32)


def _tc_mid3(s2, dinv, b2d, W3):
    return pl.pallas_call(
        _tc_mid3_body,
        grid=(_GRID,),
        in_specs=[
            pl.BlockSpec((2, _BR, 128), lambda i: (0, i, 0)),
            pl.BlockSpec((_BR, 128), lambda i: (i, 0)),
            pl.BlockSpec((1, 128), lambda i: (0, 0)),
            pl.BlockSpec((128, 128), lambda i: (0, 0)),
        ],
        out_specs=pl.BlockSpec((_BR, 128), lambda i: (i, 0)),
        out_shape=jax.ShapeDtypeStruct((NPAD, 128), jnp.float32),
    )(s2, dinv, b2d, W3)


def _tc_final_a_body(s_ref, dinv_ref, b_ref, out_ref, psum_ref):
    i = pl.program_id(0)
    dv = dinv_ref[:, 0:1]
    pre = dv * (s_ref[0] + s_ref[1]) + b_ref[...]
    out_ref[...] = pre
    rows = i * _BR + lax.broadcasted_iota(jnp.int32, (_BR, 1), 0)
    psum_ref[0] = jnp.sum(jnp.where(rows < N, pre, 0.0), axis=0, keepdims=True)


def _tc_final_a(s3, dinv, b3):
    return pl.pallas_call(
        _tc_final_a_body,
        grid=(_GRID,),
        in_specs=[
            pl.BlockSpec((2, _BR, 128), lambda i: (0, i, 0)),
            pl.BlockSpec((_BR, 128), lambda i: (i, 0)),
            pl.BlockSpec((1, 128), lambda i: (0, 0)),
        ],
        out_specs=[
            pl.BlockSpec((_BR, 128), lambda i: (i, 0)),
            pl.BlockSpec((1, 1, 128), lambda i: (i, 0, 0)),
        ],
        out_shape=[
            jax.ShapeDtypeStruct((NPAD, 128), jnp.float32),
            jax.ShapeDtypeStruct((_GRID, 1, 128), jnp.float32),
        ],
    )(s3, dinv, b3)


def _tc_final_b_body(h_ref, psum_ref, out_ref):
    total = jnp.sum(psum_ref[...])
    h = h_ref[...] / total
    t = jnp.tanh(h)
    t = t * t
    nrm = jnp.maximum(jnp.sqrt(jnp.sum(t * t, axis=1, keepdims=True)), 1e-12)
    out_ref[...] = t / nrm


def _tc_final_b(h, psum):
    return pl.pallas_call(
        _tc_final_b_body,
        grid=(_GRID,),
        in_specs=[
            pl.BlockSpec((_BR, 128), lambda i: (i, 0)),
            pl.BlockSpec((_GRID, 1, 128), lambda i: (0, 0, 0)),
        ],
        out_specs=pl.BlockSpec((_BR, 128), lambda i: (i, 0)),
        out_shape=jax.ShapeDtypeStruct((NPAD, 128), jnp.float32),
    )(h, psum)


def kernel(x, edge_index, W1, b1, W2, b2, W3, b3):
    src = edge_index[0]
    dst = edge_index[1]
    srcA = src.reshape(TILES, CH1, K1)
    srcB = srcA + NPAD
    dstM = dst.reshape(TILES, CH1, K1)
    srcE = src.reshape(32, CH2, K2)
    dstE = dst.reshape(32, CH2, K2)

    ones_full = jnp.concatenate(
        [jnp.ones((N, 128), jnp.float32), jnp.zeros((NPAD - N, 128), jnp.float32)]
    )
    zeros_full = jnp.zeros((NPAD, 128), jnp.float32)
    xp = jnp.pad(x, ((0, NPAD - N), (0, 0)))

    deg2 = _sc_degree(dstE, ones_full, zeros_full)
    dinv = _tc_dinv(deg2)

    g1 = _tc_layer1(xp, W1, dinv)
    s1 = _sc_agg1(g1.reshape(2 * NPAD, 128), srcA, srcB, dstM)
    g2 = _tc_mid2(s1, dinv, b1.reshape(2, 128), W2)
    s2 = _sc_agg2(g2, zeros_full, srcE, dstE)
    g3 = _tc_mid3(s2, dinv, b2.reshape(1, 128), W3)
    s3 = _sc_agg2(g3, zeros_full, srcE, dstE)
    h, psum = _tc_final_a(s3, dinv, b3.reshape(1, 128))
    return _tc_final_b(h, psum)[:N]


# speedup vs baseline: 13.7777x; 1.4387x over previous
import functools

import jax
import jax.numpy as jnp
from jax import lax
from jax.experimental import pallas as pl
from jax.experimental.pallas import tpu as pltpu
from jax.experimental.pallas import tpu_sc as plsc

N = 10000
E = 160000
TILES = 16
NPAD = 10112
RPT = NPAD // TILES

KA = 128
CHA = 80
KB = 128
CHB = 40
EPAD = 16 * CHA * KA

_SELU_ALPHA = 1.6732632423543772
_SELU_SCALE = 1.0507009873554805

_mesh = plsc.VectorSubcoreMesh(
    core_axis_name="c", subcore_axis_name="s", num_cores=2, num_subcores=16
)


@functools.partial(
    pl.kernel,
    out_type=jax.ShapeDtypeStruct((2, NPAD, 128), jnp.float32),
    mesh=_mesh,
    scratch_types=[
        pltpu.VMEM((CHB, KB), jnp.int32),
        pltpu.VMEM((KB, 128), jnp.float32),
        pltpu.VMEM_SHARED((NPAD, 128), jnp.float32),
    ],
)
def _sc_degree(dst_hbm, ones_hbm, zeros_hbm, out_hbm, dstv, onesv, acc):
    c = lax.axis_index("c")
    s = lax.axis_index("s")
    flat = c * TILES + s
    row0 = s * RPT
    pltpu.sync_copy(dst_hbm.at[flat], dstv)
    pltpu.sync_copy(ones_hbm.at[pl.ds(0, KB)], onesv)

    @pl.when(c == 0)
    def _():
        pltpu.sync_copy(ones_hbm.at[pl.ds(row0, RPT)], acc.at[pl.ds(row0, RPT)])

    @pl.when(c == 1)
    def _():
        pltpu.sync_copy(zeros_hbm.at[pl.ds(row0, RPT)], acc.at[pl.ds(row0, RPT)])

    plsc.subcore_barrier()

    def body(j, carry):
        pltpu.sync_copy(onesv, acc.at[dstv.at[j]], add=True)
        return carry

    lax.fori_loop(0, CHB, body, 0)
    plsc.subcore_barrier()
    pltpu.sync_copy(acc.at[pl.ds(row0, RPT)], out_hbm.at[c, pl.ds(row0, RPT)])


def _agg_edge_loop(g_hbm, acc, srcv, dstv, buf0, buf1, sg0, sg1, ch, k):
    pltpu.async_copy(g_hbm.at[srcv.at[0]], buf0, sg0)

    def body(j2, carry):
        base = 2 * j2
        pltpu.async_copy(g_hbm.at[srcv.at[base + 1]], buf1, sg1)
        pltpu.make_async_copy(g_hbm.at[pl.ds(0, k)], buf0, sg0).wait()
        pltpu.sync_copy(buf0, acc.at[dstv.at[base]], add=True)

        @pl.when(base + 2 < ch)
        def _():
            pltpu.async_copy(g_hbm.at[srcv.at[base + 2]], buf0, sg0)

        pltpu.make_async_copy(g_hbm.at[pl.ds(0, k)], buf1, sg1).wait()
        pltpu.sync_copy(buf1, acc.at[dstv.at[base + 1]], add=True)
        return carry

    lax.fori_loop(0, ch // 2, body, 0)


@functools.partial(
    pl.kernel,
    out_type=jax.ShapeDtypeStruct((2, NPAD, 128), jnp.float32),
    mesh=_mesh,
    scratch_types=[
        pltpu.VMEM((CHA, KA), jnp.int32),
        pltpu.VMEM((CHA, KA), jnp.int32),
        pltpu.VMEM((KA, 128), jnp.float32),
        pltpu.VMEM_SHARED((NPAD, 128), jnp.float32),
        pltpu.SemaphoreType.DMA,
    ],
)
def _sc_agg1(g_hbm, srcA_hbm, srcB_hbm, dst_hbm, out_hbm,
             srcv, dstv, buf0, acc, sg0):
    c = lax.axis_index("c")
    s = lax.axis_index("s")
    row0 = s * RPT

    @pl.when(c == 0)
    def _():
        pltpu.sync_copy(srcA_hbm.at[s], srcv)

    @pl.when(c == 1)
    def _():
        pltpu.sync_copy(srcB_hbm.at[s], srcv)

    pltpu.sync_copy(dst_hbm.at[s], dstv)
    pltpu.sync_copy(g_hbm.at[pl.ds(c * NPAD + row0, RPT)], acc.at[pl.ds(row0, RPT)])
    plsc.subcore_barrier()

    def body(j, carry):
        pltpu.async_copy(g_hbm.at[srcv.at[j]], buf0, sg0).wait()
        pltpu.sync_copy(buf0, acc.at[dstv.at[j]], add=True)
        return carry

    lax.fori_loop(0, CHA, body, 0)
    plsc.subcore_barrier()
    pltpu.sync_copy(acc.at[pl.ds(row0, RPT)], out_hbm.at[c, pl.ds(row0, RPT)])


@functools.partial(
    pl.kernel,
    out_type=jax.ShapeDtypeStruct((2, NPAD, 128), jnp.float32),
    mesh=_mesh,
    scratch_types=[
        pltpu.VMEM((CHB, KB), jnp.int32),
        pltpu.VMEM((CHB, KB), jnp.int32),
        pltpu.VMEM((KB, 128), jnp.float32),
        pltpu.VMEM((KB, 128), jnp.float32),
        pltpu.VMEM_SHARED((NPAD, 128), jnp.float32),
        pltpu.SemaphoreType.DMA,
        pltpu.SemaphoreType.DMA,
    ],
)
def _sc_agg2(g_hbm, zeros_hbm, src_hbm, dst_hbm, out_hbm,
             srcv, dstv, buf0, buf1, acc, sg0, sg1):
    c = lax.axis_index("c")
    s = lax.axis_index("s")
    flat = c * TILES + s
    row0 = s * RPT
    pltpu.sync_copy(src_hbm.at[flat], srcv)
    pltpu.sync_copy(dst_hbm.at[flat], dstv)

    @pl.when(c == 0)
    def _():
        pltpu.sync_copy(g_hbm.at[pl.ds(row0, RPT)], acc.at[pl.ds(row0, RPT)])

    @pl.when(c == 1)
    def _():
        pltpu.sync_copy(zeros_hbm.at[pl.ds(row0, RPT)], acc.at[pl.ds(row0, RPT)])

    plsc.subcore_barrier()
    _agg_edge_loop(g_hbm, acc, srcv, dstv, buf0, buf1, sg0, sg1, CHB, KB)
    plsc.subcore_barrier()
    pltpu.sync_copy(acc.at[pl.ds(row0, RPT)], out_hbm.at[c, pl.ds(row0, RPT)])


_BR = 632
_GRID = NPAD // _BR


def _tc_dinv_body(d_ref, out_ref):
    d = d_ref[0] + d_ref[1]
    out_ref[...] = jnp.where(d > 0, lax.rsqrt(d), 0.0)


def _tc_dinv(deg2):
    return pl.pallas_call(
        _tc_dinv_body,
        grid=(_GRID,),
        in_specs=[pl.BlockSpec((2, _BR, 128), lambda i: (0, i, 0))],
        out_specs=pl.BlockSpec((_BR, 128), lambda i: (i, 0)),
        out_shape=jax.ShapeDtypeStruct((NPAD, 128), jnp.float32),
    )(deg2)


def _tc_layer1_body(x_ref, w_ref, dinv_ref, out_ref):
    dv = dinv_ref[:, 0:1]
    g = jnp.dot(x_ref[...] * dv, w_ref[...], preferred_element_type=jnp.float32)
    out_ref[0] = g[:, :128]
    out_ref[1] = g[:, 128:]


def _tc_layer1(x, W1, dinv):
    return pl.pallas_call(
        _tc_layer1_body,
        grid=(_GRID,),
        in_specs=[
            pl.BlockSpec((_BR, 256), lambda i: (i, 0)),
            pl.BlockSpec((256, 256), lambda i: (0, 0)),
            pl.BlockSpec((_BR, 128), lambda i: (i, 0)),
        ],
        out_specs=pl.BlockSpec((2, _BR, 128), lambda i: (0, i, 0)),
        out_shape=jax.ShapeDtypeStruct((2, NPAD, 128), jnp.float32),
    )(x, W1, dinv)


def _selu(x):
    return _SELU_SCALE * jnp.where(x > 0, x, _SELU_ALPHA * jnp.exp(x) - _SELU_ALPHA)


def _tc_mid2_body(s_ref, dinv_ref, b_ref, w_ref, out_ref):
    dv = dinv_ref[:, 0:1]
    u0 = dv * _selu(dv * s_ref[0] + b_ref[0:1, :])
    u1 = dv * _selu(dv * s_ref[1] + b_ref[1:2, :])
    g = jnp.dot(u0, w_ref[:128, :], preferred_element_type=jnp.float32)
    g += jnp.dot(u1, w_ref[128:, :], preferred_element_type=jnp.float32)
    out_ref[...] = g


def _tc_mid2(s1, dinv, b2d, W2):
    return pl.pallas_call(
        _tc_mid2_body,
        grid=(_GRID,),
        in_specs=[
            pl.BlockSpec((2, _BR, 128), lambda i: (0, i, 0)),
            pl.BlockSpec((_BR, 128), lambda i: (i, 0)),
            pl.BlockSpec((2, 128), lambda i: (0, 0)),
            pl.BlockSpec((256, 128), lambda i: (0, 0)),
        ],
        out_specs=pl.BlockSpec((_BR, 128), lambda i: (i, 0)),
        out_shape=jax.ShapeDtypeStruct((NPAD, 128), jnp.float32),
    )(s1, dinv, b2d, W2)


def _tc_mid3_body(s_ref, dinv_ref, b_ref, w_ref, out_ref):
    dv = dinv_ref[:, 0:1]
    u = dv * _selu(dv * (s_ref[0] + s_ref[1]) + b_ref[...])
    out_ref[...] = jnp.dot(u, w_ref[...], preferred_element_type=jnp.float32)


def _tc_mid3(s2, dinv, b2d, W3):
    return pl.pallas_call(
        _tc_mid3_body,
        grid=(_GRID,),
        in_specs=[
            pl.BlockSpec((2, _BR, 128), lambda i: (0, i, 0)),
            pl.BlockSpec((_BR, 128), lambda i: (i, 0)),
            pl.BlockSpec((1, 128), lambda i: (0, 0)),
            pl.BlockSpec((128, 128), lambda i: (0, 0)),
        ],
        out_specs=pl.BlockSpec((_BR, 128), lambda i: (i, 0)),
        out_shape=jax.ShapeDtypeStruct((NPAD, 128), jnp.float32),
    )(s2, dinv, b2d, W3)


def _tc_final_a_body(s_ref, dinv_ref, b_ref, out_ref, psum_ref):
    i = pl.program_id(0)
    dv = dinv_ref[:, 0:1]
    pre = dv * (s_ref[0] + s_ref[1]) + b_ref[...]
    out_ref[...] = pre
    rows = i * _BR + lax.broadcasted_iota(jnp.int32, (_BR, 1), 0)
    psum_ref[0] = jnp.sum(jnp.where(rows < N, pre, 0.0), axis=0, keepdims=True)


def _tc_final_a(s3, dinv, b3):
    return pl.pallas_call(
        _tc_final_a_body,
        grid=(_GRID,),
        in_specs=[
            pl.BlockSpec((2, _BR, 128), lambda i: (0, i, 0)),
            pl.BlockSpec((_BR, 128), lambda i: (i, 0)),
            pl.BlockSpec((1, 128), lambda i: (0, 0)),
        ],
        out_specs=[
            pl.BlockSpec((_BR, 128), lambda i: (i, 0)),
            pl.BlockSpec((1, 1, 128), lambda i: (i, 0, 0)),
        ],
        out_shape=[
            jax.ShapeDtypeStruct((NPAD, 128), jnp.float32),
            jax.ShapeDtypeStruct((_GRID, 1, 128), jnp.float32),
        ],
    )(s3, dinv, b3)


def _tc_final_b_body(h_ref, psum_ref, out_ref):
    total = jnp.sum(psum_ref[...])
    h = h_ref[...] / total
    t = jnp.tanh(h)
    t = t * t
    nrm = jnp.maximum(jnp.sqrt(jnp.sum(t * t, axis=1, keepdims=True)), 1e-12)
    out_ref[...] = t / nrm


def _tc_final_b(h, psum):
    return pl.pallas_call(
        _tc_final_b_body,
        grid=(_GRID,),
        in_specs=[
            pl.BlockSpec((_BR, 128), lambda i: (i, 0)),
            pl.BlockSpec((_GRID, 1, 128), lambda i: (0, 0, 0)),
        ],
        out_specs=pl.BlockSpec((_BR, 128), lambda i: (i, 0)),
        out_shape=jax.ShapeDtypeStruct((NPAD, 128), jnp.float32),
    )(h, psum)


def kernel(x, edge_index, W1, b1, W2, b2, W3, b3):
    pad_idx = N + (jnp.arange(EPAD - E, dtype=jnp.int32) % (NPAD - N))
    srcp = jnp.concatenate([edge_index[0], pad_idx])
    dstp = jnp.concatenate([edge_index[1], pad_idx])
    srcA = srcp.reshape(TILES, CHA, KA)
    srcB = srcA + NPAD
    dstM = dstp.reshape(TILES, CHA, KA)
    srcE = srcp.reshape(32, CHB, KB)
    dstE = dstp.reshape(32, CHB, KB)

    ones_full = jnp.concatenate(
        [jnp.ones((N, 128), jnp.float32), jnp.zeros((NPAD - N, 128), jnp.float32)]
    )
    zeros_full = jnp.zeros((NPAD, 128), jnp.float32)
    xp = jnp.pad(x, ((0, NPAD - N), (0, 0)))

    deg2 = _sc_degree(dstE, ones_full, zeros_full)
    dinv = _tc_dinv(deg2)

    g1 = _tc_layer1(xp, W1, dinv)
    s1 = _sc_agg1(g1.reshape(2 * NPAD, 128), srcA, srcB, dstM)
    g2 = _tc_mid2(s1, dinv, b1.reshape(2, 128), W2)
    s2 = _sc_agg2(g2, zeros_full, srcE, dstE)
    g3 = _tc_mid3(s2, dinv, b2.reshape(1, 128), W3)
    s3 = _sc_agg2(g3, zeros_full, srcE, dstE)
    h, psum = _tc_final_a(s3, dinv, b3.reshape(1, 128))
    return _tc_final_b(h, psum)[:N]
